# Initial kernel scaffold; baseline (speedup 1.0000x reference)
import jax
import jax.numpy as jnp
from jax.experimental import pallas as pl


def kernel(input_ids, targets, token_embedding_table):
    raise NotImplementedError("write your pallas kernel here")


# speedup vs baseline: 2.2366x; 2.2366x over previous
import functools

import jax
import jax.numpy as jnp
from jax import lax
from jax.experimental import pallas as pl
from jax.experimental.pallas import tpu as pltpu
from jax.experimental.pallas import tpu_sc as plsc

V = 1000
VP = 1024
NC = 2
NS = 16
L = 16
NW = NC * NS


def _lse_body(tab_ref, out_ref, copy_ref):
    x = tab_ref[...]
    m = jnp.max(x, axis=1, keepdims=True)
    s = jnp.sum(jnp.exp(x - m), axis=1, keepdims=True)
    out_ref[...] = m + jnp.log(s)
    copy_ref[...] = x


@functools.lru_cache(maxsize=None)
def _make_sc_gather(B):
    SPW = B // NW
    CH = 32
    NCH = SPW // CH
    mesh = plsc.VectorSubcoreMesh(core_axis_name="c", subcore_axis_name="s")

    @functools.partial(
        pl.kernel,
        mesh=mesh,
        compiler_params=pltpu.CompilerParams(
            use_tc_tiling_on_sc=True, disable_bounds_checks=True),
        out_type=jax.ShapeDtypeStruct((B, V), jnp.float32),
        scratch_types=[
            pltpu.VMEM((SPW,), jnp.int32),
            pltpu.VMEM((CH, 8, 128), jnp.float32),
            pltpu.VMEM((CH, 8, 128), jnp.float32),
            pltpu.VMEM((CH, 128), jnp.float32),
            pltpu.VMEM((CH, 128), jnp.float32),
            pltpu.SemaphoreType.DMA,
            pltpu.SemaphoreType.DMA,
            pltpu.SemaphoreType.DMA,
            pltpu.SemaphoreType.DMA,
        ],
    )
    def sc_gather(tab3_hbm, ids_hbm, out_hbm,
                  ids_v, rows0, rows1, stage0, stage1, g0, g1, w0, w1):
        wid = lax.axis_index("s") * NC + lax.axis_index("c")
        base = wid * SPW
        pltpu.sync_copy(ids_hbm.at[pl.ds(base, SPW)], ids_v)

        def start_gather(c, rows_b, gsem):
            h = pltpu.make_async_copy(
                tab3_hbm.at[ids_v.at[pl.ds(c * CH, CH)]], rows_b, gsem)
            h.start()
            return h

        def write_chunk(c, rows_b, stage_b, wsem):
            r0 = base + c * CH
            hs = []
            zero = wid * 0
            for t in range(8):
                h = pltpu.make_async_copy(
                    rows_b.at[:, t, :],
                    out_hbm.at[pl.ds(r0, CH), pl.ds(zero + t * 128, 128)],
                    wsem)
                h.start()
                hs.append(h)
            return hs

        def body(g, carry):
            c0 = 2 * g
            c1 = 2 * g + 1
            hg0 = start_gather(c0, rows0, g0)
            hg1 = start_gather(c1, rows1, g1)
            hg0.wait()
            hw0 = write_chunk(c0, rows0, stage0, w0)
            hg1.wait()
            hw1 = write_chunk(c1, rows1, stage1, w1)
            for h in hw0:
                h.wait()
            for h in hw1:
                h.wait()
            return carry

        lax.fori_loop(0, NCH // 2, body, 0)

    return sc_gather


@functools.lru_cache(maxsize=None)
def _make_sc_loss(B):
    SPW = B // NW
    GC = 80
    NG = SPW // GC
    mesh = plsc.VectorSubcoreMesh(core_axis_name="c", subcore_axis_name="s")

    @functools.partial(
        pl.kernel,
        mesh=mesh,
        compiler_params=pltpu.CompilerParams(use_tc_tiling_on_sc=False),
        out_type=jax.ShapeDtypeStruct((NW, L), jnp.float32),
        scratch_types=[
            pltpu.VMEM((SPW,), jnp.int32),
            pltpu.VMEM((SPW,), jnp.int32),
            pltpu.VMEM((SPW,), jnp.float32),
            pltpu.VMEM((SPW,), jnp.float32),
            pltpu.VMEM((L,), jnp.float32),
            pltpu.SemaphoreType.DMA,
        ],
    )
    def sc_loss(tabflat_hbm, ids_hbm, tgt_hbm, lse_hbm, part_hbm,
                ids_v, flat_v, lse_b, tv_b, acc_v, a):
        wid = lax.axis_index("s") * NC + lax.axis_index("c")
        base = wid * SPW
        pltpu.sync_copy(ids_hbm.at[pl.ds(base, SPW)], ids_v)
        pltpu.sync_copy(tgt_hbm.at[pl.ds(base, SPW)], flat_v)
        acc_v[...] = jnp.zeros((L,), jnp.float32)

        def flatten_idx(i, carry):
            sl = pl.ds(i * L, L)
            flat_v[sl] = flat_v[sl] + ids_v[sl] * V
            return carry

        lax.fori_loop(0, SPW // L, flatten_idx, 0)

        handles = []
        for gidx in range(NG):
            sl = pl.ds(gidx * GC, GC)
            hl = pltpu.make_async_copy(
                lse_hbm.at[ids_v.at[sl]], lse_b.at[sl], a)
            hl.start()
            ht = pltpu.make_async_copy(
                tabflat_hbm.at[flat_v.at[sl]], tv_b.at[sl], a)
            ht.start()
            handles.append(hl)
            handles.append(ht)
        for h in handles:
            h.wait()

        def accum(i, carry):
            sl = pl.ds(i * L, L)
            acc_v[...] = acc_v[...] + (lse_b[sl] - tv_b[sl])
            return carry

        lax.fori_loop(0, SPW // L, accum, 0)
        pltpu.sync_copy(acc_v, part_hbm.at[wid])

    return sc_loss


def kernel(input_ids, targets, token_embedding_table):
    B = input_ids.shape[0] * input_ids.shape[1]
    ids = input_ids.reshape(B).astype(jnp.int32)
    tgs = targets.reshape(B).astype(jnp.int32)
    tab3 = jnp.pad(token_embedding_table,
                   ((0, 0), (0, VP - V))).reshape(V, 8, 128)
    lse, tabcopy = pl.pallas_call(
        _lse_body,
        out_shape=[
            jax.ShapeDtypeStruct((V, 1), jnp.float32),
            jax.ShapeDtypeStruct((V, V), jnp.float32),
        ],
    )(token_embedding_table)
    logits = _make_sc_gather(B)(tab3, ids)
    parts = _make_sc_loss(B)(
        tabcopy.reshape(V * V), ids, tgs, lse.reshape(V))
    loss = jnp.sum(parts) / B
    return logits, loss
